# baseline (speedup 1.0000x reference)
import jax
import jax.numpy as jnp
from jax import lax
from jax.experimental import pallas as pl
from jax.experimental.pallas import tpu as pltpu
from jax.experimental.pallas import tpu_sc as plsc

_Q = 16
_K = 4096
_D = 768
_C = 512
_CS = 32
_HALF = _K // 2
_N_STRUCT = 17
_N_BC0 = 3
_N_BC1 = 4
_NS = 16


def _body(emb_hbm, out_hbm, zidx_v, sidx_v, rows_a, bcast_sh, struct_sh,
          gsem, bsem, wsem):
    cid = lax.axis_index("c")
    sid = lax.axis_index("s")
    i = sid
    half = (sid + cid) & 1
    base = i * _K
    s_end = ((i + 513 + 7) >> 3) << 3

    for q in range(2):
        zidx_v[pl.ds(q * 16, 16)] = jnp.zeros((16,), jnp.int32)
    pltpu.async_copy(emb_hbm.at[zidx_v], rows_a, gsem).wait()
    pltpu.sync_copy(
        rows_a, bcast_sh.at[pl.ds(pl.multiple_of(sid * 32, 8), 32)])
    plsc.subcore_barrier()

    pend = []
    for k in range(_N_BC0):
        j0 = jnp.where(
            half == 0,
            jnp.minimum(s_end + _C * k, _HALF - _C),
            jnp.minimum(_HALF + _C * k, _K - _C),
        )
        pend.append(
            pltpu.async_copy(
                bcast_sh,
                out_hbm.at[pl.ds(pl.multiple_of(base + j0, 8), _C)], bsem))

    @pl.when(half == 1)
    def _():
        extra = [
            pltpu.async_copy(
                bcast_sh,
                out_hbm.at[pl.ds(
                    pl.multiple_of(
                        base + jnp.minimum(_HALF + _C * k, _K - _C), 8),
                    _C)], bsem)
            for k in range(_N_BC0, _N_BC1)
        ]
        for p in extra:
            p.wait()

    @pl.when(half == 0)
    def _():
        def slot(p):
            off = pl.multiple_of((sid * 2 + p) * _CS, 8)
            return struct_sh.at[pl.ds(off, _CS)]

        def fill_idx(k):
            j0 = jnp.minimum(_CS * k, s_end - _CS)
            top = i + 512 - j0
            for q in range(_CS // 16):
                sidx_v[pl.ds(q * 16, 16)] = jnp.maximum(
                    (top - q * 16) - lax.iota(jnp.int32, 16), 0)
            return j0

        writes = []
        for k in range(_N_STRUCT):
            j0 = fill_idx(k)
            pltpu.async_copy(emb_hbm.at[sidx_v], rows_a, gsem).wait()
            if k >= 2:
                writes[k - 2].wait()
            pltpu.sync_copy(rows_a, slot(k % 2))
            writes.append(
                pltpu.async_copy(
                    slot(k % 2),
                    out_hbm.at[pl.ds(pl.multiple_of(base + j0, 8), _CS)],
                    wsem))
        for w in writes[-2:]:
            w.wait()

    for p in pend:
        p.wait()


@jax.jit
def _rpe(emb_weight):
    mesh = plsc.VectorSubcoreMesh(core_axis_name="c", subcore_axis_name="s")
    run = pl.kernel(
        _body,
        out_type=jax.ShapeDtypeStruct((_Q * _K, _D), jnp.float32),
        mesh=mesh,
        scratch_types=[
            pltpu.VMEM((32,), jnp.int32),
            pltpu.VMEM((_CS,), jnp.int32),
            pltpu.VMEM((_CS, _D), jnp.float32),
            pltpu.VMEM_SHARED((_C, _D), jnp.float32),
            pltpu.VMEM_SHARED((_NS * 2 * _CS, _D), jnp.float32),
            pltpu.SemaphoreType.DMA,
            pltpu.SemaphoreType.DMA,
            pltpu.SemaphoreType.DMA,
        ],
    )
    return run(emb_weight).reshape(_Q, _K, _D)


def kernel(q_len, k_len, emb_weight):
    return _rpe(emb_weight)


# speedup vs baseline: 1.2151x; 1.2151x over previous
import jax
import jax.numpy as jnp
from jax import lax
from jax.experimental import pallas as pl
from jax.experimental.pallas import tpu as pltpu
from jax.experimental.pallas import tpu_sc as plsc

_Q = 16
_K = 4096
_D = 768
_C = 512
_CS = 32
_HALF = _K // 2
_N_STRUCT = 17
_N_BC0 = 3
_N_BC1 = 4
_NS = 16


def _body(emb_hbm, out_hbm, zidx_v, sidx_v, rows_a, rows_b, bcast_sh,
          gsem, bsem, wsem):
    cid = lax.axis_index("c")
    sid = lax.axis_index("s")
    i = sid
    half = (sid + cid) & 1
    base = i * _K
    s_end = ((i + 513 + 7) >> 3) << 3

    for q in range(2):
        zidx_v[pl.ds(q * 16, 16)] = jnp.zeros((16,), jnp.int32)
    pltpu.async_copy(emb_hbm.at[zidx_v], rows_a, gsem).wait()
    pltpu.sync_copy(
        rows_a, bcast_sh.at[pl.ds(pl.multiple_of(sid * 32, 8), 32)])
    plsc.subcore_barrier()

    pend = []
    for k in range(_N_BC0):
        j0 = jnp.where(
            half == 0,
            jnp.minimum(s_end + _C * k, _HALF - _C),
            jnp.minimum(_HALF + _C * k, _K - _C),
        )
        pend.append(
            pltpu.async_copy(
                bcast_sh,
                out_hbm.at[pl.ds(pl.multiple_of(base + j0, 8), _C)], bsem))

    @pl.when(half == 1)
    def _():
        extra = [
            pltpu.async_copy(
                bcast_sh,
                out_hbm.at[pl.ds(
                    pl.multiple_of(
                        base + jnp.minimum(_HALF + _C * k, _K - _C), 8),
                    _C)], bsem)
            for k in range(_N_BC0, _N_BC1)
        ]
        for p in extra:
            p.wait()

    @pl.when(half == 0)
    def _():
        tbufs = (rows_a, rows_b)

        def fill_idx(k):
            j0 = jnp.minimum(_CS * k, s_end - _CS)
            top = i + 512 - j0
            for q in range(_CS // 16):
                sidx_v[pl.ds(q * 16, 16)] = jnp.maximum(
                    (top - q * 16) - lax.iota(jnp.int32, 16), 0)
            return j0

        writes = []
        for k in range(_N_STRUCT):
            j0 = fill_idx(k)
            pltpu.async_copy(emb_hbm.at[sidx_v], tbufs[k % 2], gsem).wait()
            if k >= 2:
                writes[k - 2].wait()
            writes.append(
                pltpu.async_copy(
                    tbufs[k % 2],
                    out_hbm.at[pl.ds(pl.multiple_of(base + j0, 8), _CS)],
                    wsem))
        for w in writes[-2:]:
            w.wait()

    for p in pend:
        p.wait()


@jax.jit
def _rpe(emb_weight):
    mesh = plsc.VectorSubcoreMesh(core_axis_name="c", subcore_axis_name="s")
    run = pl.kernel(
        _body,
        out_type=jax.ShapeDtypeStruct((_Q * _K, _D), jnp.float32),
        mesh=mesh,
        scratch_types=[
            pltpu.VMEM((32,), jnp.int32),
            pltpu.VMEM((_CS,), jnp.int32),
            pltpu.VMEM((_CS, _D), jnp.float32),
            pltpu.VMEM((_CS, _D), jnp.float32),
            pltpu.VMEM_SHARED((_C, _D), jnp.float32),
            pltpu.SemaphoreType.DMA,
            pltpu.SemaphoreType.DMA,
            pltpu.SemaphoreType.DMA,
        ],
    )
    return run(emb_weight).reshape(_Q, _K, _D)


def kernel(q_len, k_len, emb_weight):
    return _rpe(emb_weight)
